# baseline (speedup 1.0000x reference)
import functools

import jax
import jax.numpy as jnp
from jax import lax
from jax.experimental import pallas as pl
from jax.experimental.pallas import tpu as pltpu
from jax.experimental.pallas import tpu_sc as plsc

B = 16384
F = 26
K = 16
NC = 2
NS = 16
NW = NC * NS
SPW = B // NW
CH = 64
NCHUNK = SPW // CH
RPC = CH * F
GW = 128
NJ = RPC // GW
XROWS = B * F // GW


def _fm_body(xs_hbm, wf_hbm, v_hbm, w0_hbm, out_hbm,
             idx0, idx1, vrow0, vrow1, wrow0, wrow1,
             w0buf, outbuf, sem0, sem1):
  wid = lax.axis_index("s") * NC + lax.axis_index("c")
  base = pl.multiple_of(wid * SPW, SPW)
  xbase = pl.multiple_of(wid * (SPW * F), SPW * F)

  idxb = (idx0, idx1)
  vrowb = (vrow0, vrow1)
  wrowb = (wrow0, wrow1)
  semb = (sem0, sem1)

  pltpu.sync_copy(w0_hbm, w0buf)
  w0vec = w0buf[...]
  iota = lax.iota(jnp.int32, 16)

  def fire(c):
    b = c % 2
    pltpu.sync_copy(xs_hbm.at[pl.ds(xbase + c * RPC, RPC)], idxb[b])
    hs = []
    for j in range(NJ):
      idxr = idxb[b].at[pl.ds(j * GW, GW)]
      hs.append(pltpu.async_copy(
          v_hbm.at[idxr], vrowb[b].at[pl.ds(j * GW, GW)], semb[b]))
      hs.append(pltpu.async_copy(
          wf_hbm.at[idxr], wrowb[b].at[pl.ds(j * GW, GW)], semb[b]))
    return hs

  perms = tuple(iota ^ st for st in (1, 2, 4, 8))

  def compute(c):
    b = c % 2
    vr = vrowb[b]
    wr = wrowb[b]

    def group_body(g, _):
      inter = jnp.zeros((16,), jnp.float32)
      for s in range(16):
        sl = g * 16 + s
        row = vr[sl]
        acc = row
        acc2 = row * row
        for f in range(1, F):
          row = vr[f * CH + sl]
          acc = acc + row
          acc2 = acc2 + row * row
        t = acc * acc - acc2
        for p in perms:
          t = t + t[p]
        inter = jnp.where(iota == s, t, inter)
      lin = wr[pl.ds(g * 16, 16)]
      for f in range(1, F):
        lin = lin + wr[pl.ds(f * CH + g * 16, 16)]
      z = w0vec + lin + 0.5 * inter
      sig = 1.0 / (1.0 + jnp.exp(-z))
      outbuf[pl.ds(c * CH + g * 16, 16)] = sig
      return 0

    lax.fori_loop(0, CH // 16, group_body, 0)

  handles = {}
  handles[0] = fire(0)
  for c in range(NCHUNK):
    if c + 1 < NCHUNK:
      handles[c + 1] = fire(c + 1)
    for h in handles.pop(c):
      h.wait()
    compute(c)

  pltpu.sync_copy(outbuf, out_hbm.at[pl.ds(base, SPW)])


N_FEAT = 1000000
TBLK = 8192
T0 = (N_FEAT // TBLK) * TBLK
TR = N_FEAT - T0


NB = T0 // TBLK + 1


def _pack_body(vt_ref, vtail_ref, w_ref, out_ref, wf_ref):
  i = pl.program_id(0)
  last = (i == NB - 1).astype(jnp.float32)
  src = vt_ref[...] + last * (vtail_ref[...] - vt_ref[...])
  t = src.T
  t3 = t.reshape(TBLK // 8, 8, K)
  out_ref[...] = jnp.concatenate([t3[:, s, :] for s in range(8)], axis=1)
  wf_ref[...] = w_ref[...][:, 0]


def _tc_pack(vt, vtail, w):
  return pl.pallas_call(
      _pack_body,
      grid=(NB,),
      in_specs=[
          pl.BlockSpec((K, TBLK), lambda i: (0, jnp.minimum(i, NB - 2))),
          pl.BlockSpec((K, TBLK), lambda i: (0, 0)),
          pl.BlockSpec((TBLK, 1), lambda i: (i, 0)),
      ],
      out_specs=[
          pl.BlockSpec((TBLK // 8, 128), lambda i: (i, 0)),
          pl.BlockSpec((TBLK,), lambda i: (i,)),
      ],
      out_shape=[
          jax.ShapeDtypeStruct((N_FEAT // 8, 128), jnp.float32),
          jax.ShapeDtypeStruct((N_FEAT,), jnp.float32),
      ],
  )(vt, vtail, w)


@jax.jit
def _fm(xs2, wf, v, w0v):
  mesh = plsc.VectorSubcoreMesh(core_axis_name="c", subcore_axis_name="s")
  body = functools.partial(
      pl.kernel,
      out_type=jax.ShapeDtypeStruct((B,), jnp.float32),
      mesh=mesh,
      scratch_types=[
          pltpu.VMEM((RPC,), jnp.int32),
          pltpu.VMEM((RPC,), jnp.int32),
          pltpu.VMEM((RPC, K), jnp.float32),
          pltpu.VMEM((RPC, K), jnp.float32),
          pltpu.VMEM((RPC,), jnp.float32),
          pltpu.VMEM((RPC,), jnp.float32),
          pltpu.VMEM((16,), jnp.float32),
          pltpu.VMEM((SPW,), jnp.float32),
          pltpu.SemaphoreType.DMA,
          pltpu.SemaphoreType.DMA,
      ],
      compiler_params=pltpu.CompilerParams(use_tc_tiling_on_sc=False),
  )(_fm_body)
  return body(xs2, wf, v, w0v)


def kernel(x_sparse, x_dense, w0, w, v):
  del x_dense
  xs2 = x_sparse.reshape(B // CH, CH, F).transpose(0, 2, 1).reshape(-1)
  vt = v.T
  vtailpad = jnp.pad(vt[:, T0:], ((0, 0), (0, TBLK - TR)))
  vpack, wf = _tc_pack(vt, vtailpad, w)
  vrm = vpack.reshape(N_FEAT, K)
  w0v = jnp.broadcast_to(w0, (16,))
  return _fm(xs2, wf, vrm, w0v)


# speedup vs baseline: 2.2251x; 2.2251x over previous
import functools

import jax
import jax.numpy as jnp
from jax import lax
from jax.experimental import pallas as pl
from jax.experimental.pallas import tpu as pltpu
from jax.experimental.pallas import tpu_sc as plsc

B = 16384
F = 26
K = 16
NC = 2
NS = 16
NW = NC * NS
SPW = B // NW
CH = 64
NCHUNK = SPW // CH
RPC = CH * F
GW = 128
NJ = RPC // GW
XROWS = B * F // GW


def _fm_body(xs_hbm, wf_hbm, v_hbm, w0_hbm, out_hbm,
             idx0, idx1, vrow0, vrow1, wrow0, wrow1,
             w0buf, outbuf, sem0, sem1):
  wid = lax.axis_index("s") * NC + lax.axis_index("c")
  base = pl.multiple_of(wid * SPW, SPW)
  xbase = pl.multiple_of(wid * (SPW * F), SPW * F)

  idxb = (idx0, idx1)
  vrowb = (vrow0, vrow1)
  wrowb = (wrow0, wrow1)
  semb = (sem0, sem1)

  pltpu.sync_copy(w0_hbm, w0buf)
  w0vec = w0buf[...]
  iota = lax.iota(jnp.int32, 16)

  def fire(c):
    b = c % 2
    pltpu.sync_copy(xs_hbm.at[pl.ds(xbase + c * RPC, RPC)], idxb[b])
    hs = []
    for j in range(NJ):
      idxr = idxb[b].at[pl.ds(j * GW, GW)]
      hs.append(pltpu.async_copy(
          v_hbm.at[idxr], vrowb[b].at[pl.ds(j * GW, GW)], semb[b]))
      hs.append(pltpu.async_copy(
          wf_hbm.at[idxr], wrowb[b].at[pl.ds(j * GW, GW)], semb[b]))
    return hs

  perms = tuple(iota ^ st for st in (1, 2, 4, 8))

  def compute(c):
    b = c % 2
    vr = vrowb[b]
    wr = wrowb[b]

    def group_body(g, _):
      inter = jnp.zeros((16,), jnp.float32)
      for s in range(16):
        sl = g * 16 + s
        row = vr[sl]
        acc = row
        acc2 = row * row
        for f in range(1, F):
          row = vr[f * CH + sl]
          acc = acc + row
          acc2 = acc2 + row * row
        t = acc * acc - acc2
        for p in perms:
          t = t + t[p]
        inter = jnp.where(iota == s, t, inter)
      lin = wr[pl.ds(g * 16, 16)]
      for f in range(1, F):
        lin = lin + wr[pl.ds(f * CH + g * 16, 16)]
      z = w0vec + lin + 0.5 * inter
      sig = 1.0 / (1.0 + jnp.exp(-z))
      outbuf[pl.ds(c * CH + g * 16, 16)] = sig
      return 0

    lax.fori_loop(0, CH // 16, group_body, 0)

  handles = {}
  handles[0] = fire(0)
  for c in range(NCHUNK):
    if c + 1 < NCHUNK:
      handles[c + 1] = fire(c + 1)
    for h in handles.pop(c):
      h.wait()
    compute(c)

  pltpu.sync_copy(outbuf, out_hbm.at[pl.ds(base, SPW)])


N_FEAT = 1000000
TBLK = 8192
T0 = (N_FEAT // TBLK) * TBLK
TR = N_FEAT - T0


NB = T0 // TBLK + 1


def _pack_body(vt_ref, vtail_ref, out_ref):
  i = pl.program_id(0)
  last = (i == NB - 1).astype(jnp.float32)
  src = vt_ref[...] + last * (vtail_ref[...] - vt_ref[...])
  t = src.T
  t3 = t.reshape(TBLK // 8, 8, K)
  for s in range(8):
    out_ref[:, pl.ds(16 * s, 16)] = t3[:, s, :]


def _tc_pack(vt, vtail):
  return pl.pallas_call(
      _pack_body,
      grid=(NB,),
      in_specs=[
          pl.BlockSpec((K, TBLK), lambda i: (0, jnp.minimum(i, NB - 2))),
          pl.BlockSpec((K, TBLK), lambda i: (0, 0)),
      ],
      out_specs=pl.BlockSpec((TBLK // 8, 128), lambda i: (i, 0)),
      out_shape=jax.ShapeDtypeStruct((N_FEAT // 8, 128), jnp.float32),
  )(vt, vtail)


@jax.jit
def _fm(xs2, wf, v, w0v):
  mesh = plsc.VectorSubcoreMesh(core_axis_name="c", subcore_axis_name="s")
  body = functools.partial(
      pl.kernel,
      out_type=jax.ShapeDtypeStruct((B,), jnp.float32),
      mesh=mesh,
      scratch_types=[
          pltpu.VMEM((RPC,), jnp.int32),
          pltpu.VMEM((RPC,), jnp.int32),
          pltpu.VMEM((RPC, K), jnp.float32),
          pltpu.VMEM((RPC, K), jnp.float32),
          pltpu.VMEM((RPC,), jnp.float32),
          pltpu.VMEM((RPC,), jnp.float32),
          pltpu.VMEM((16,), jnp.float32),
          pltpu.VMEM((SPW,), jnp.float32),
          pltpu.SemaphoreType.DMA,
          pltpu.SemaphoreType.DMA,
      ],
      compiler_params=pltpu.CompilerParams(use_tc_tiling_on_sc=False),
  )(_fm_body)
  return body(xs2, wf, v, w0v)


def kernel(x_sparse, x_dense, w0, w, v):
  del x_dense
  xs2 = x_sparse.reshape(B // CH, CH, F).transpose(0, 2, 1).reshape(-1)
  vt = v.T
  vtailpad = jnp.pad(vt[:, T0:], ((0, 0), (0, TBLK - TR)))
  vrm = _tc_pack(vt, vtailpad).reshape(N_FEAT, K)
  wf = w.reshape(-1)
  w0v = jnp.broadcast_to(w0, (16,))
  return _fm(xs2, wf, vrm, w0v)
